# baseline (speedup 1.0000x reference)
import functools
import itertools

import numpy as np
import jax
import jax.numpy as jnp
from jax import lax
from jax.experimental import pallas as pl
from jax.experimental.pallas import tpu as pltpu
from jax.experimental.pallas import tpu_sc as plsc

N = 24
DIM = 3
THRESH = 0.5
MAXTRI = 1000
NE = 276
NT = 2024
NQ = 10626
NTP = 2048
NEP = 288
NQP = 10752
FCH, FCL = 8, 256
QCH, QCL = 8, 1344
FG = FCL // 16
QG = QCL // 16

XE_OFF = 2048
XN_OFF = 2624
ZE_OFF = 2048
PRN_OFF = 2624
YN_OFF = 2656
EM_OFF = 2048
DIN_OFF = 2624
WEC_OFF = 2048

POOL_DIFF = {2: 3, 4: 6, 8: 9}
SNAP = (1, 2, 4)


def _build_tables():
    tri = np.array(list(itertools.combinations(range(N), 3)), dtype=np.int64)
    tet = np.array(list(itertools.combinations(range(N), 4)), dtype=np.int64)
    pair = np.array(list(itertools.combinations(range(N), 2)), dtype=np.int64)
    L2 = np.zeros((N, N), dtype=np.int64)
    for idx, (i, j) in enumerate(pair):
        L2[i, j] = idx
    L3 = np.zeros((N, N, N), dtype=np.int64)
    for idx, (i, j, k) in enumerate(tri):
        L3[i, j, k] = idx
    L4 = {tuple(t): q for q, t in enumerate(tet)}
    tf = np.zeros((NQ, 4), dtype=np.int32)
    for q, (i, j, k, l) in enumerate(tet):
        tf[q] = [L3[j, k, l], L3[i, k, l], L3[i, j, l], L3[i, j, k]]
    tt = np.zeros((NT, 21), dtype=np.int32)
    for t, (i, j, k) in enumerate(tri):
        ms = [m for m in range(N) if m not in (i, j, k)]
        tt[t] = [L4[tuple(sorted((i, j, k, m)))] for m in ms]
    te = np.zeros((NT, 3), dtype=np.int32)
    for t, (i, j, k) in enumerate(tri):
        te[t] = [L2[j, k], L2[i, k], L2[i, j]]
    et = np.zeros((NE, 22), dtype=np.int32)
    for e, (i, j) in enumerate(pair):
        ms = [m for m in range(N) if m not in (i, j)]
        et[e] = [L3[tuple(sorted((i, j, m)))] for m in ms]
    tt_p = np.zeros((21, NTP), dtype=np.int32)
    tt_p[:, :NT] = tt.T
    tt_chunks = np.stack([tt_p[:, c * FCL:(c + 1) * FCL].reshape(-1)
                          for c in range(FCH)])
    tf_p = np.zeros((4, NQP), dtype=np.int32)
    tf_p[:, :NQ] = tf.T
    tf_chunks = np.stack([tf_p[:, c * QCL:(c + 1) * QCL].reshape(-1)
                          for c in range(QCH)])
    et_p = np.zeros((22, NEP), dtype=np.int32)
    et_p[:, :NE] = et.T
    te_p = np.zeros((3, NTP), dtype=np.int32)
    te_p[:, :NT] = te.T
    return tri, tet, tt_chunks, tf_chunks, et_p.reshape(-1), te_p.reshape(-1)


(_TRI_NP, _TET_NP, _TTC, _TFC, _ETF, _TEF) = _build_tables()
_IIX = np.repeat(np.arange(N), N)
_JJX = np.tile(np.arange(N), N)


def _row(r):
    return jnp.full((16,), r, jnp.int32)


def _sc_body(a0_h, c0_h, tt_h, tf_h, a1_h, c1_h, d1_h, et_h, te_h, out_h,
             A, B, C, D, Z, IT, FT, ACC, STG, sem0, sem1):
    c = lax.axis_index("c")
    s = lax.axis_index("s")
    zero = jnp.zeros((16,), jnp.float32)
    for r in range(48):
        ACC[pl.ds(r * 16, 16)] = zero

    def acc_add(r, v):
        ACC[pl.ds(r * 16, 16)] = ACC[pl.ds(r * 16, 16)] + v

    def colsum(base):
        v = zero
        for j in range(16):
            v = v + plsc.load_gather(
                ACC, [lax.iota(jnp.int32, 16) * 16 + (base * 16 + j)])
        return v

    @pl.when(c == 0)
    def _tet():
        pltpu.sync_copy(a0_h.at[s], A)
        pltpu.sync_copy(c0_h.at[s], C)

        def dbuf_sweep(tab_h, nch, words, buf, ngrp, body):
            def cp(cc, half, sem):
                return pltpu.make_async_copy(
                    tab_h.at[cc], buf.at[pl.ds(half * words, words)], sem)
            cp(0, 0, sem0).start()

            def pair(i, _):
                c0 = 2 * i
                cp(c0 + 1, 1, sem1).start()
                cp(c0, 0, sem0).wait()

                def grp0(gg, _):
                    body(0, c0, gg)
                    return 0
                lax.fori_loop(0, ngrp, grp0, 0)

                @pl.when(i < nch // 2 - 1)
                def _pre():
                    cp(c0 + 2, 0, sem0).start()
                cp(c0 + 1, 1, sem1).wait()

                def grp1(gg, _):
                    body(1, c0 + 1, gg)
                    return 0
                lax.fori_loop(0, ngrp, grp1, 0)
                return 0
            lax.fori_loop(0, nch // 2, pair, 0)

        def tt_sweep(gather_fn, store_fn):
            def body(half, cc, gg):
                base = half * (21 * FCL)
                accs = None
                for k in range(21):
                    idx = IT[pl.ds(base + k * FCL + gg * 16, 16)]
                    vals = gather_fn(idx)
                    accs = vals if accs is None else [a + v for a, v in zip(accs, vals)]
                store_fn(cc * FG + gg, accs)
            dbuf_sweep(tt_h, FCH, 21 * FCL, IT, FG, body)

        def tf_sweep(nrows, upd_fn):
            def body(half, cq, gg):
                base = half * (4 * QCL)
                sl = [zero] * nrows
                for f in range(4):
                    idx = FT[pl.ds(base + f * QCL + gg * 16, 16)]
                    for r in range(nrows):
                        sl[r] = sl[r] + plsc.load_gather(Z, [idx + (r * NTP)])
                upd_fn(cq * QG + gg, sl)
            dbuf_sweep(tf_h, QCH, 4 * QCL, FT, QG, body)

        def zstore0(fg, accs):
            Z[pl.ds(fg * 16, 16)] = accs[0]
        tt_sweep(lambda idx: [plsc.load_gather(C, [idx])], zstore0)

        def deg_upd(tq, sl):
            qvv = C[pl.ds(tq * 16, 16)]
            deg = qvv * (sl[0] - 4.0 * qvv)
            D[pl.ds(tq * 16, 16)] = jnp.where(deg > 0.0, 1.0 / deg, 0.0)
            acc_add(15, qvv)
            for d in range(DIM):
                acc_add(d, qvv * A[pl.ds(d * NQP + tq * 16, 16)])
        tf_sweep(1, deg_upd)

        def step(k, _):
            def zgather(idx):
                w = plsc.load_gather(D, [idx])
                return [w * plsc.load_gather(A, [idx + (d * NQP)])
                        for d in range(DIM)]

            def zstore(fg, accs):
                for d in range(DIM):
                    Z[pl.ds(d * NTP + fg * 16, 16)] = accs[d]
            tt_sweep(zgather, zstore)

            is_diff = jnp.logical_or(k == 2, jnp.logical_or(k == 4, k == 8))
            is_snap = jnp.logical_or(k == 1, jnp.logical_or(k == 2, k == 4))
            bi = jnp.where(k == 2, 3, jnp.where(k == 4, 6, 9))

            def upd(tq, sl):
                qvv = C[pl.ds(tq * 16, 16)]
                w = D[pl.ds(tq * 16, 16)]
                for d in range(DIM):
                    xv = A[pl.ds(d * NQP + tq * 16, 16)]
                    nv = 0.5 * xv + 0.5 * qvv * (sl[d] - 4.0 * w * xv)
                    A[pl.ds(d * NQP + tq * 16, 16)] = nv

                    @pl.when(is_diff)
                    def _diff():
                        dv = jnp.abs(B[pl.ds(d * NQP + tq * 16, 16)] - nv)
                        r = (bi + d) * 16
                        ACC[pl.ds(r, 16)] = ACC[pl.ds(r, 16)] + qvv * dv

                    @pl.when(k == 8)
                    def _cur():
                        acc_add(12 + d, qvv * nv)

                    @pl.when(is_snap)
                    def _snap():
                        B[pl.ds(d * NQP + tq * 16, 16)] = nv
            tf_sweep(DIM, upd)
            return 0
        lax.fori_loop(1, 9, step, 0)

        STG[pl.ds(48, 16)] = colsum(0)
        pltpu.sync_copy(STG.at[pl.ds(48, 16)], out_h.at[s, pl.ds(48, 16)])

    @pl.when(c == 1)
    def _tri():
        pltpu.sync_copy(a1_h.at[s], A)
        pltpu.sync_copy(c1_h.at[s], C)
        pltpu.sync_copy(d1_h.at[s], D)
        pltpu.sync_copy(et_h, IT.at[pl.ds(0, 22 * NEP)])
        pltpu.sync_copy(te_h, FT.at[pl.ds(0, 3 * NTP)])

        ACC[pl.ds(31 * 16, 16)] = jnp.full((16,), 1.5, jnp.float32)

        def egrp(gg, _):
            emv = C[pl.ds(EM_OFF + gg * 16, 16)]
            acc_add(47, emv)
            for d in range(DIM):
                xev = A[pl.ds(d * NQP + XE_OFF + gg * 16, 16)]
                axv = jnp.abs(xev)
                acc_add(32 + d, emv * xev)
                acc_add(35 + d, 0.25 * emv * axv)
                acc_add(38 + d, 0.1875 * emv * axv)
                acc_add(41 + d, 0.05859375 * emv * axv)
                acc_add(44 + d, (0.5 ** 8) * emv * xev)
            return 0
        lax.fori_loop(0, 36, egrp, 0)

        def et_sweep(gather_fn, store_fn):
            def grp(gg, _):
                accs = None
                for m in range(22):
                    idx = IT[pl.ds(m * NEP + gg * 16, 16)]
                    vals = gather_fn(idx)
                    accs = vals if accs is None else [a + v for a, v in zip(accs, vals)]
                store_fn(gg, accs)
                return 0
            lax.fori_loop(0, NEP // 16, grp, 0)

        def te_sweep(nrows, upd_fn):
            def grp(gg, _):
                sl = [zero] * nrows
                for f in range(3):
                    idx = FT[pl.ds(f * NTP + gg * 16, 16)]
                    for r in range(nrows):
                        sl[r] = sl[r] + plsc.load_gather(
                            B, [idx + (r * NQP + ZE_OFF)])
                upd_fn(gg, sl)
                return 0
            lax.fori_loop(0, NTP // 16, grp, 0)

        def zstore0(gg, accs):
            B[pl.ds(ZE_OFF + gg * 16, 16)] = accs[0]
        et_sweep(lambda idx: [plsc.load_gather(C, [idx])], zstore0)

        def tdeg_upd(tg, sl):
            tvv = C[pl.ds(tg * 16, 16)]
            deg = 2.0 * tvv * (sl[0] - 3.0 * tvv)
            D[pl.ds(tg * 16, 16)] = jnp.where(deg > 0.0, 1.0 / deg, 0.0)
            acc_add(15, tvv)
            for d in range(DIM):
                acc_add(d, tvv * A[pl.ds(d * NQP + tg * 16, 16)])
        te_sweep(1, tdeg_upd)

        for gg in range(2):
            for d in range(DIM):
                acc_add(16 + d, A[pl.ds(d * NQP + XN_OFF + gg * 16, 16)])

        for k in range(1, 9):
            def zgather(idx):
                w = plsc.load_gather(D, [idx])
                return [w * plsc.load_gather(A, [idx + (d * NQP)])
                        for d in range(DIM)]

            def zstore(gg, accs):
                for d in range(DIM):
                    B[pl.ds(d * NQP + ZE_OFF + gg * 16, 16)] = accs[d]
            et_sweep(zgather, zstore)

            def upd(tg, sl, k=k):
                tvv = C[pl.ds(tg * 16, 16)]
                w = D[pl.ds(tg * 16, 16)]
                for d in range(DIM):
                    xv = A[pl.ds(d * NQP + tg * 16, 16)]
                    nv = 0.5 * xv + tvv * (sl[d] - 3.0 * w * xv)
                    A[pl.ds(d * NQP + tg * 16, 16)] = nv
                    if k in POOL_DIFF:
                        dv = jnp.abs(B[pl.ds(d * NQP + tg * 16, 16)] - nv)
                        acc_add(POOL_DIFF[k] + d, tvv * dv)
                    if k == 8:
                        acc_add(12 + d, tvv * nv)
                    if k in SNAP:
                        B[pl.ds(d * NQP + tg * 16, 16)] = nv
            te_sweep(DIM, upd)

            for gg in range(2):
                dinv_ = C[pl.ds(DIN_OFF + gg * 16, 16)]
                for d in range(DIM):
                    B[pl.ds(d * NQP + YN_OFF + gg * 16, 16)] = (
                        dinv_ * A[pl.ds(d * NQP + XN_OFF + gg * 16, 16)])
            for d in range(DIM):
                def jloop(j, carry, d=d):
                    a0, a1 = carry
                    y = plsc.load_gather(
                        B, [jnp.full((16,), d * NQP + YN_OFF, jnp.int32) + j])
                    a0 = a0 + y * D[pl.ds(WEC_OFF + j * 32, 16)]
                    a1 = a1 + y * D[pl.ds(WEC_OFF + j * 32 + 16, 16)]
                    return (a0, a1)
                an0, an1 = lax.fori_loop(0, N, jloop, (zero, zero))
                for gg, an in ((0, an0), (1, an1)):
                    xv = A[pl.ds(d * NQP + XN_OFF + gg * 16, 16)]
                    nv = 0.5 * (xv + an)
                    A[pl.ds(d * NQP + XN_OFF + gg * 16, 16)] = nv
                    if k in POOL_DIFF:
                        dv = jnp.abs(
                            B[pl.ds(d * NQP + PRN_OFF + gg * 16, 16)] - nv)
                        acc_add(16 + POOL_DIFF[k] + d, dv)
                    if k == 8:
                        acc_add(16 + 12 + d, nv)
                    if k in SNAP:
                        B[pl.ds(d * NQP + PRN_OFF + gg * 16, 16)] = nv

        STG[pl.ds(0, 16)] = colsum(16)
        STG[pl.ds(16, 16)] = colsum(32)
        STG[pl.ds(32, 16)] = colsum(0)
        pltpu.sync_copy(STG.at[pl.ds(0, 48)], out_h.at[s, pl.ds(0, 48)])


_sc_call = functools.partial(
    pl.kernel,
    out_type=jax.ShapeDtypeStruct((16, 64), jnp.float32),
    mesh=plsc.VectorSubcoreMesh(core_axis_name="c", subcore_axis_name="s"),
    compiler_params=pltpu.CompilerParams(needs_layout_passes=False),
    scratch_types=[
        pltpu.VMEM((DIM * NQP,), jnp.float32),
        pltpu.VMEM((DIM * NQP,), jnp.float32),
        pltpu.VMEM((NQP,), jnp.float32),
        pltpu.VMEM((NQP,), jnp.float32),
        pltpu.VMEM((DIM * NTP,), jnp.float32),
        pltpu.VMEM((2 * 21 * FCL,), jnp.int32),
        pltpu.VMEM((2 * 4 * QCL,), jnp.int32),
        pltpu.VMEM((64 * 16,), jnp.float32),
        pltpu.VMEM((64,), jnp.float32),
        pltpu.SemaphoreType.DMA,
        pltpu.SemaphoreType.DMA,
    ],
)(_sc_body)


def kernel(point_clouds, alphas, sigma):
    pc = point_clouds.astype(jnp.float32)
    al = alphas.astype(jnp.float32)
    sig = jnp.asarray(sigma, dtype=jnp.float32)
    B_, Np, Dm = pc.shape
    G = B_ * al.shape[0]

    Xs, Wes = [], []
    for p in range(B_):
        for w in range(al.shape[0]):
            Xg = pc[p] * al[w]
            Gg = Xg @ Xg.T
            dgg = jnp.diag(Gg)
            Dg = dgg[None, :] + dgg[:, None] - 2.0 * Gg
            Wmg = jnp.exp(-Dg / sig)
            Xs.append(Xg)
            Wes.append(Wmg)
    X = jnp.stack(Xs)
    Wm = jnp.stack(Wes)
    adj = (Wm >= THRESH).astype(jnp.float32)
    We = Wm * adj
    em = adj.reshape(G, Np * Np)

    t0, t1, t2 = _TRI_NP[:, 0], _TRI_NP[:, 1], _TRI_NP[:, 2]
    af = em
    tvr = af[:, t0 * Np + t1] * af[:, t0 * Np + t2] * af[:, t1 * Np + t2]
    cnt = jnp.cumsum(tvr.astype(jnp.int32), axis=1)
    tv = tvr * (cnt <= MAXTRI).astype(jnp.float32)
    q0, q1, q2, q3 = (_TET_NP[:, 0], _TET_NP[:, 1], _TET_NP[:, 2], _TET_NP[:, 3])
    qv = (af[:, q0 * Np + q1] * af[:, q0 * Np + q2] * af[:, q0 * Np + q3]
          * af[:, q1 * Np + q2] * af[:, q1 * Np + q3] * af[:, q2 * Np + q3])

    Xe = 0.5 * (X[:, _IIX, :] + X[:, _JJX, :])
    Xt = (X[:, t0, :] + X[:, t1, :] + X[:, t2, :]) / 3.0
    Xq = (X[:, q0, :] + X[:, q1, :] + X[:, q2, :] + X[:, q3, :]) / 4.0

    deg_n = We.sum(axis=1)
    deg_n = deg_n.at[0].add(We.sum(axis=(0, 1)))
    di_n = jnp.where(deg_n > 0, 1.0 / deg_n, 0.0)
    wec = jnp.transpose(We, (0, 2, 1))
    wec = wec.at[0].add(We.sum(axis=0).T)
    wec = jnp.pad(wec, ((0, 0), (0, 0), (0, 32 - Np))).reshape(G, N * 32)

    def pad_to(x, n):
        return jnp.pad(x, ((0, 0), (0, 0), (0, n - x.shape[-1])))

    a0 = pad_to(jnp.transpose(Xq, (0, 2, 1)), NQP).reshape(G, DIM * NQP)
    c0 = jnp.pad(qv, ((0, 0), (0, NQP - NQ)))
    a1 = jnp.concatenate([
        pad_to(jnp.transpose(Xt, (0, 2, 1)), NTP),
        jnp.transpose(Xe, (0, 2, 1)),
        pad_to(jnp.transpose(X, (0, 2, 1)), 32),
        jnp.zeros((G, DIM, NQP - XN_OFF - 32), jnp.float32),
    ], axis=-1).reshape(G, DIM * NQP)
    c1 = jnp.concatenate([
        jnp.pad(tv, ((0, 0), (0, NTP - NT))), em,
        jnp.pad(di_n, ((0, 0), (0, 32 - Np))),
        jnp.zeros((G, NQP - DIN_OFF - 32), jnp.float32),
    ], axis=-1)
    d1 = jnp.concatenate([
        jnp.zeros((G, WEC_OFF), jnp.float32), wec,
        jnp.zeros((G, NQP - WEC_OFF - N * 32), jnp.float32),
    ], axis=-1)

    out = _sc_call(a0, c0, jnp.asarray(_TTC), jnp.asarray(_TFC),
                   a1, c1, d1, jnp.asarray(_ETF), jnp.asarray(_TEF))

    out = out.reshape(G, 4, 16)
    sums = out[:, :, :15].sum(axis=1)
    cnts = out[:, :, 15].sum(axis=1)
    pooled = sums / jnp.maximum(cnts, 1.0)[:, None]
    return pooled.reshape(B_, -1)


# speedup vs baseline: 1.1168x; 1.1168x over previous
import functools
import itertools

import numpy as np
import jax
import jax.numpy as jnp
from jax import lax
from jax.experimental import pallas as pl
from jax.experimental.pallas import tpu as pltpu
from jax.experimental.pallas import tpu_sc as plsc

N = 24
DIM = 3
THRESH = 0.5
MAXTRI = 1000
NE = 276
NT = 2024
NQ = 10626
NTP = 2048
NEP = 288
NQP = 10752
FCH, FCL = 4, 512
QCH, QCL = 4, 2688
FG = FCL // 16
QG = QCL // 16

XE_OFF = 2048
XN_OFF = 2624
ZE_OFF = 2048
PRN_OFF = 2624
YN_OFF = 2656
EM_OFF = 2048
DIN_OFF = 2624
WEC_OFF = 2048

POOL_DIFF = {2: 3, 4: 6, 8: 9}
SNAP = (1, 2, 4)


def _build_tables():
    tri = np.array(list(itertools.combinations(range(N), 3)), dtype=np.int64)
    tet = np.array(list(itertools.combinations(range(N), 4)), dtype=np.int64)
    pair = np.array(list(itertools.combinations(range(N), 2)), dtype=np.int64)
    L2 = np.zeros((N, N), dtype=np.int64)
    for idx, (i, j) in enumerate(pair):
        L2[i, j] = idx
    L3 = np.zeros((N, N, N), dtype=np.int64)
    for idx, (i, j, k) in enumerate(tri):
        L3[i, j, k] = idx
    L4 = {tuple(t): q for q, t in enumerate(tet)}
    tf = np.zeros((NQ, 4), dtype=np.int32)
    for q, (i, j, k, l) in enumerate(tet):
        tf[q] = [L3[j, k, l], L3[i, k, l], L3[i, j, l], L3[i, j, k]]
    tt = np.zeros((NT, 21), dtype=np.int32)
    for t, (i, j, k) in enumerate(tri):
        ms = [m for m in range(N) if m not in (i, j, k)]
        tt[t] = [L4[tuple(sorted((i, j, k, m)))] for m in ms]
    te = np.zeros((NT, 3), dtype=np.int32)
    for t, (i, j, k) in enumerate(tri):
        te[t] = [L2[j, k], L2[i, k], L2[i, j]]
    et = np.zeros((NE, 22), dtype=np.int32)
    for e, (i, j) in enumerate(pair):
        ms = [m for m in range(N) if m not in (i, j)]
        et[e] = [L3[tuple(sorted((i, j, m)))] for m in ms]
    tt_p = np.zeros((21, NTP), dtype=np.int32)
    tt_p[:, :NT] = tt.T
    tt_chunks = np.stack([tt_p[:, c * FCL:(c + 1) * FCL].reshape(-1)
                          for c in range(FCH)])
    tf_p = np.zeros((4, NQP), dtype=np.int32)
    tf_p[:, :NQ] = tf.T
    tf_chunks = np.stack([tf_p[:, c * QCL:(c + 1) * QCL].reshape(-1)
                          for c in range(QCH)])
    et_p = np.zeros((22, NEP), dtype=np.int32)
    et_p[:, :NE] = et.T
    te_p = np.zeros((3, NTP), dtype=np.int32)
    te_p[:, :NT] = te.T
    return tri, tet, tt_chunks, tf_chunks, et_p.reshape(-1), te_p.reshape(-1)


(_TRI_NP, _TET_NP, _TTC, _TFC, _ETF, _TEF) = _build_tables()
_IIX = np.repeat(np.arange(N), N)
_JJX = np.tile(np.arange(N), N)


def _row(r):
    return jnp.full((16,), r, jnp.int32)


def _sc_body(a0_h, c0_h, tt_h, tf_h, a1_h, c1_h, d1_h, et_h, te_h, out_h,
             A, B, C, D, Z, IT, FT, ACC, STG, sem0, sem1):
    c = lax.axis_index("c")
    s = lax.axis_index("s")
    zero = jnp.zeros((16,), jnp.float32)
    for r in range(48):
        ACC[pl.ds(r * 16, 16)] = zero

    def acc_add(r, v):
        ACC[pl.ds(r * 16, 16)] = ACC[pl.ds(r * 16, 16)] + v

    def colsum(base):
        v = zero
        for j in range(16):
            v = v + plsc.load_gather(
                ACC, [lax.iota(jnp.int32, 16) * 16 + (base * 16 + j)])
        return v

    @pl.when(c == 0)
    def _tet():
        pltpu.sync_copy(a0_h.at[s], A)
        pltpu.sync_copy(c0_h.at[s], C)

        def tt_sweep(gather_fn, store_fn):
            def chunk(cc, _):
                pltpu.sync_copy(tt_h.at[cc], IT.at[pl.ds(0, 21 * FCL)])

                def grp(gg, _):
                    accs = None
                    for k in range(21):
                        idx = IT[pl.ds(k * FCL + gg * 16, 16)]
                        vals = gather_fn(idx)
                        accs = vals if accs is None else [a + v for a, v in zip(accs, vals)]
                    store_fn(cc * FG + gg, accs)
                    return 0
                lax.fori_loop(0, FG, grp, 0)
                return 0
            lax.fori_loop(0, FCH, chunk, 0)

        def tf_sweep(nrows, upd_fn):
            def chunk(cq, _):
                pltpu.sync_copy(tf_h.at[cq], FT.at[pl.ds(0, 4 * QCL)])

                def grp(gg, _):
                    sl = [zero] * nrows
                    for f in range(4):
                        idx = FT[pl.ds(f * QCL + gg * 16, 16)]
                        for r in range(nrows):
                            sl[r] = sl[r] + plsc.load_gather(Z, [idx + (r * NTP)])
                    upd_fn(cq * QG + gg, sl)
                    return 0
                lax.fori_loop(0, QG, grp, 0)
                return 0
            lax.fori_loop(0, QCH, chunk, 0)

        def zstore0(fg, accs):
            Z[pl.ds(fg * 16, 16)] = accs[0]
        tt_sweep(lambda idx: [plsc.load_gather(C, [idx])], zstore0)

        def deg_upd(tq, sl):
            qvv = C[pl.ds(tq * 16, 16)]
            deg = qvv * (sl[0] - 4.0 * qvv)
            D[pl.ds(tq * 16, 16)] = jnp.where(deg > 0.0, 1.0 / deg, 0.0)
            acc_add(15, qvv)
            for d in range(DIM):
                acc_add(d, qvv * A[pl.ds(d * NQP + tq * 16, 16)])
        tf_sweep(1, deg_upd)

        for k in range(1, 9):
            def upd(tq, sl, k=k):
                qvv = C[pl.ds(tq * 16, 16)]
                w = D[pl.ds(tq * 16, 16)]
                for d in range(DIM):
                    xv = A[pl.ds(d * NQP + tq * 16, 16)]
                    nv = 0.5 * xv + 0.5 * qvv * (sl[d] - 4.0 * w * xv)
                    A[pl.ds(d * NQP + tq * 16, 16)] = nv
                    if k in POOL_DIFF:
                        dv = jnp.abs(B[pl.ds(d * NQP + tq * 16, 16)] - nv)
                        acc_add(POOL_DIFF[k] + d, qvv * dv)
                    if k == 8:
                        acc_add(12 + d, qvv * nv)
                    if k in SNAP:
                        B[pl.ds(d * NQP + tq * 16, 16)] = nv

            def zgather(idx):
                w = plsc.load_gather(D, [idx])
                return [w * plsc.load_gather(A, [idx + (d * NQP)])
                        for d in range(DIM)]

            def zstore(fg, accs):
                for d in range(DIM):
                    Z[pl.ds(d * NTP + fg * 16, 16)] = accs[d]
            tt_sweep(zgather, zstore)
            tf_sweep(DIM, upd)

        STG[pl.ds(48, 16)] = colsum(0)
        pltpu.sync_copy(STG.at[pl.ds(48, 16)], out_h.at[s, pl.ds(48, 16)])

    @pl.when(c == 1)
    def _tri():
        pltpu.sync_copy(a1_h.at[s], A)
        pltpu.sync_copy(c1_h.at[s], C)
        pltpu.sync_copy(d1_h.at[s], D)
        pltpu.sync_copy(et_h, IT.at[pl.ds(0, 22 * NEP)])
        pltpu.sync_copy(te_h, FT.at[pl.ds(0, 3 * NTP)])

        ACC[pl.ds(31 * 16, 16)] = jnp.full((16,), 1.5, jnp.float32)

        def egrp(gg, _):
            emv = C[pl.ds(EM_OFF + gg * 16, 16)]
            acc_add(47, emv)
            for d in range(DIM):
                xev = A[pl.ds(d * NQP + XE_OFF + gg * 16, 16)]
                axv = jnp.abs(xev)
                acc_add(32 + d, emv * xev)
                acc_add(35 + d, 0.25 * emv * axv)
                acc_add(38 + d, 0.1875 * emv * axv)
                acc_add(41 + d, 0.05859375 * emv * axv)
                acc_add(44 + d, (0.5 ** 8) * emv * xev)
            return 0
        lax.fori_loop(0, 36, egrp, 0)

        def et_sweep(gather_fn, store_fn):
            def grp(gg, _):
                accs = None
                for m in range(22):
                    idx = IT[pl.ds(m * NEP + gg * 16, 16)]
                    vals = gather_fn(idx)
                    accs = vals if accs is None else [a + v for a, v in zip(accs, vals)]
                store_fn(gg, accs)
                return 0
            lax.fori_loop(0, NEP // 16, grp, 0)

        def te_sweep(nrows, upd_fn):
            def grp(gg, _):
                sl = [zero] * nrows
                for f in range(3):
                    idx = FT[pl.ds(f * NTP + gg * 16, 16)]
                    for r in range(nrows):
                        sl[r] = sl[r] + plsc.load_gather(
                            B, [idx + (r * NQP + ZE_OFF)])
                upd_fn(gg, sl)
                return 0
            lax.fori_loop(0, NTP // 16, grp, 0)

        def zstore0(gg, accs):
            B[pl.ds(ZE_OFF + gg * 16, 16)] = accs[0]
        et_sweep(lambda idx: [plsc.load_gather(C, [idx])], zstore0)

        def tdeg_upd(tg, sl):
            tvv = C[pl.ds(tg * 16, 16)]
            deg = 2.0 * tvv * (sl[0] - 3.0 * tvv)
            D[pl.ds(tg * 16, 16)] = jnp.where(deg > 0.0, 1.0 / deg, 0.0)
            acc_add(15, tvv)
            for d in range(DIM):
                acc_add(d, tvv * A[pl.ds(d * NQP + tg * 16, 16)])
        te_sweep(1, tdeg_upd)

        for gg in range(2):
            for d in range(DIM):
                acc_add(16 + d, A[pl.ds(d * NQP + XN_OFF + gg * 16, 16)])

        for k in range(1, 9):
            def zgather(idx):
                w = plsc.load_gather(D, [idx])
                return [w * plsc.load_gather(A, [idx + (d * NQP)])
                        for d in range(DIM)]

            def zstore(gg, accs):
                for d in range(DIM):
                    B[pl.ds(d * NQP + ZE_OFF + gg * 16, 16)] = accs[d]
            et_sweep(zgather, zstore)

            def upd(tg, sl, k=k):
                tvv = C[pl.ds(tg * 16, 16)]
                w = D[pl.ds(tg * 16, 16)]
                for d in range(DIM):
                    xv = A[pl.ds(d * NQP + tg * 16, 16)]
                    nv = 0.5 * xv + tvv * (sl[d] - 3.0 * w * xv)
                    A[pl.ds(d * NQP + tg * 16, 16)] = nv
                    if k in POOL_DIFF:
                        dv = jnp.abs(B[pl.ds(d * NQP + tg * 16, 16)] - nv)
                        acc_add(POOL_DIFF[k] + d, tvv * dv)
                    if k == 8:
                        acc_add(12 + d, tvv * nv)
                    if k in SNAP:
                        B[pl.ds(d * NQP + tg * 16, 16)] = nv
            te_sweep(DIM, upd)

            for gg in range(2):
                dinv_ = C[pl.ds(DIN_OFF + gg * 16, 16)]
                for d in range(DIM):
                    B[pl.ds(d * NQP + YN_OFF + gg * 16, 16)] = (
                        dinv_ * A[pl.ds(d * NQP + XN_OFF + gg * 16, 16)])
            for d in range(DIM):
                def jloop(j, carry, d=d):
                    a0, a1 = carry
                    y = plsc.load_gather(
                        B, [jnp.full((16,), d * NQP + YN_OFF, jnp.int32) + j])
                    a0 = a0 + y * D[pl.ds(WEC_OFF + j * 32, 16)]
                    a1 = a1 + y * D[pl.ds(WEC_OFF + j * 32 + 16, 16)]
                    return (a0, a1)
                an0, an1 = lax.fori_loop(0, N, jloop, (zero, zero))
                for gg, an in ((0, an0), (1, an1)):
                    xv = A[pl.ds(d * NQP + XN_OFF + gg * 16, 16)]
                    nv = 0.5 * (xv + an)
                    A[pl.ds(d * NQP + XN_OFF + gg * 16, 16)] = nv
                    if k in POOL_DIFF:
                        dv = jnp.abs(
                            B[pl.ds(d * NQP + PRN_OFF + gg * 16, 16)] - nv)
                        acc_add(16 + POOL_DIFF[k] + d, dv)
                    if k == 8:
                        acc_add(16 + 12 + d, nv)
                    if k in SNAP:
                        B[pl.ds(d * NQP + PRN_OFF + gg * 16, 16)] = nv

        STG[pl.ds(0, 16)] = colsum(16)
        STG[pl.ds(16, 16)] = colsum(32)
        STG[pl.ds(32, 16)] = colsum(0)
        pltpu.sync_copy(STG.at[pl.ds(0, 48)], out_h.at[s, pl.ds(0, 48)])


_sc_call = functools.partial(
    pl.kernel,
    out_type=jax.ShapeDtypeStruct((16, 64), jnp.float32),
    mesh=plsc.VectorSubcoreMesh(core_axis_name="c", subcore_axis_name="s"),
    compiler_params=pltpu.CompilerParams(needs_layout_passes=False),
    scratch_types=[
        pltpu.VMEM((DIM * NQP,), jnp.float32),
        pltpu.VMEM((DIM * NQP,), jnp.float32),
        pltpu.VMEM((NQP,), jnp.float32),
        pltpu.VMEM((NQP,), jnp.float32),
        pltpu.VMEM((DIM * NTP,), jnp.float32),
        pltpu.VMEM((21 * FCL,), jnp.int32),
        pltpu.VMEM((4 * QCL,), jnp.int32),
        pltpu.VMEM((64 * 16,), jnp.float32),
        pltpu.VMEM((64,), jnp.float32),
        pltpu.SemaphoreType.DMA,
        pltpu.SemaphoreType.DMA,
    ],
)(_sc_body)


def kernel(point_clouds, alphas, sigma):
    pc = point_clouds.astype(jnp.float32)
    al = alphas.astype(jnp.float32)
    sig = jnp.asarray(sigma, dtype=jnp.float32)
    B_, Np, Dm = pc.shape
    G = B_ * al.shape[0]

    Xs, Wes = [], []
    for p in range(B_):
        for w in range(al.shape[0]):
            Xg = pc[p] * al[w]
            Gg = Xg @ Xg.T
            dgg = jnp.diag(Gg)
            Dg = dgg[None, :] + dgg[:, None] - 2.0 * Gg
            Wmg = jnp.exp(-Dg / sig)
            Xs.append(Xg)
            Wes.append(Wmg)
    X = jnp.stack(Xs)
    Wm = jnp.stack(Wes)
    adj = (Wm >= THRESH).astype(jnp.float32)
    We = Wm * adj
    em = adj.reshape(G, Np * Np)

    t0, t1, t2 = _TRI_NP[:, 0], _TRI_NP[:, 1], _TRI_NP[:, 2]
    af = em
    tvr = af[:, t0 * Np + t1] * af[:, t0 * Np + t2] * af[:, t1 * Np + t2]
    cnt = jnp.cumsum(tvr.astype(jnp.int32), axis=1)
    tv = tvr * (cnt <= MAXTRI).astype(jnp.float32)
    q0, q1, q2, q3 = (_TET_NP[:, 0], _TET_NP[:, 1], _TET_NP[:, 2], _TET_NP[:, 3])
    qv = (af[:, q0 * Np + q1] * af[:, q0 * Np + q2] * af[:, q0 * Np + q3]
          * af[:, q1 * Np + q2] * af[:, q1 * Np + q3] * af[:, q2 * Np + q3])

    Xe = 0.5 * (X[:, _IIX, :] + X[:, _JJX, :])
    Xt = (X[:, t0, :] + X[:, t1, :] + X[:, t2, :]) / 3.0
    Xq = (X[:, q0, :] + X[:, q1, :] + X[:, q2, :] + X[:, q3, :]) / 4.0

    deg_n = We.sum(axis=1)
    deg_n = deg_n.at[0].add(We.sum(axis=(0, 1)))
    di_n = jnp.where(deg_n > 0, 1.0 / deg_n, 0.0)
    wec = jnp.transpose(We, (0, 2, 1))
    wec = wec.at[0].add(We.sum(axis=0).T)
    wec = jnp.pad(wec, ((0, 0), (0, 0), (0, 32 - Np))).reshape(G, N * 32)

    def pad_to(x, n):
        return jnp.pad(x, ((0, 0), (0, 0), (0, n - x.shape[-1])))

    a0 = pad_to(jnp.transpose(Xq, (0, 2, 1)), NQP).reshape(G, DIM * NQP)
    c0 = jnp.pad(qv, ((0, 0), (0, NQP - NQ)))
    a1 = jnp.concatenate([
        pad_to(jnp.transpose(Xt, (0, 2, 1)), NTP),
        jnp.transpose(Xe, (0, 2, 1)),
        pad_to(jnp.transpose(X, (0, 2, 1)), 32),
        jnp.zeros((G, DIM, NQP - XN_OFF - 32), jnp.float32),
    ], axis=-1).reshape(G, DIM * NQP)
    c1 = jnp.concatenate([
        jnp.pad(tv, ((0, 0), (0, NTP - NT))), em,
        jnp.pad(di_n, ((0, 0), (0, 32 - Np))),
        jnp.zeros((G, NQP - DIN_OFF - 32), jnp.float32),
    ], axis=-1)
    d1 = jnp.concatenate([
        jnp.zeros((G, WEC_OFF), jnp.float32), wec,
        jnp.zeros((G, NQP - WEC_OFF - N * 32), jnp.float32),
    ], axis=-1)

    out = _sc_call(a0, c0, jnp.asarray(_TTC), jnp.asarray(_TFC),
                   a1, c1, d1, jnp.asarray(_ETF), jnp.asarray(_TEF))

    out = out.reshape(G, 4, 16)
    sums = out[:, :, :15].sum(axis=1)
    cnts = out[:, :, 15].sum(axis=1)
    pooled = sums / jnp.maximum(cnts, 1.0)[:, None]
    return pooled.reshape(B_, -1)


# speedup vs baseline: 1.1439x; 1.0242x over previous
import functools
import itertools

import numpy as np
import jax
import jax.numpy as jnp
from jax import lax
from jax.experimental import pallas as pl
from jax.experimental.pallas import tpu as pltpu
from jax.experimental.pallas import tpu_sc as plsc

N = 24
DIM = 3
THRESH = 0.5
MAXTRI = 1000
NE = 276
NT = 2024
NQ = 10626
NTP = 2048
NEP = 288
NQP = 10752
FCH, FCL = 4, 512
QCH, QCL = 4, 2688
FG = FCL // 16
QG = QCL // 16

XE_OFF = 2048
XN_OFF = 2624
ZE_OFF = 2048
PRN_OFF = 2624
YN_OFF = 2656
EM_OFF = 2048
DIN_OFF = 2624
WEC_OFF = 2048

POOL_DIFF = {2: 3, 4: 6, 8: 9}
SNAP = (1, 2, 4)


def _build_tables():
    tri = np.array(list(itertools.combinations(range(N), 3)), dtype=np.int64)
    tet = np.array(list(itertools.combinations(range(N), 4)), dtype=np.int64)
    pair = np.array(list(itertools.combinations(range(N), 2)), dtype=np.int64)
    L2 = np.zeros((N, N), dtype=np.int64)
    for idx, (i, j) in enumerate(pair):
        L2[i, j] = idx
    L3 = np.zeros((N, N, N), dtype=np.int64)
    for idx, (i, j, k) in enumerate(tri):
        L3[i, j, k] = idx
    L4 = {tuple(t): q for q, t in enumerate(tet)}
    tf = np.zeros((NQ, 4), dtype=np.int32)
    for q, (i, j, k, l) in enumerate(tet):
        tf[q] = [L3[j, k, l], L3[i, k, l], L3[i, j, l], L3[i, j, k]]
    tt = np.zeros((NT, 21), dtype=np.int32)
    for t, (i, j, k) in enumerate(tri):
        ms = [m for m in range(N) if m not in (i, j, k)]
        tt[t] = [L4[tuple(sorted((i, j, k, m)))] for m in ms]
    te = np.zeros((NT, 3), dtype=np.int32)
    for t, (i, j, k) in enumerate(tri):
        te[t] = [L2[j, k], L2[i, k], L2[i, j]]
    et = np.zeros((NE, 22), dtype=np.int32)
    for e, (i, j) in enumerate(pair):
        ms = [m for m in range(N) if m not in (i, j)]
        et[e] = [L3[tuple(sorted((i, j, m)))] for m in ms]
    tt_p = np.zeros((21, NTP), dtype=np.int32)
    tt_p[:, :NT] = tt.T
    tt_chunks = np.stack([tt_p[:, c * FCL:(c + 1) * FCL].reshape(-1)
                          for c in range(FCH)])
    tf_p = np.zeros((4, NQP), dtype=np.int32)
    tf_p[:, :NQ] = tf.T
    tf_chunks = np.stack([tf_p[:, c * QCL:(c + 1) * QCL].reshape(-1)
                          for c in range(QCH)])
    et_p = np.zeros((22, NEP), dtype=np.int32)
    et_p[:, :NE] = et.T
    te_p = np.zeros((3, NTP), dtype=np.int32)
    te_p[:, :NT] = te.T
    return tri, tet, tt_chunks, tf_chunks, et_p.reshape(-1), te_p.reshape(-1)


(_TRI_NP, _TET_NP, _TTC, _TFC, _ETF, _TEF) = _build_tables()
_IIX = np.repeat(np.arange(N), N)
_JJX = np.tile(np.arange(N), N)


def _row(r):
    return jnp.full((16,), r, jnp.int32)


def _sc_body(a0_h, c0_h, tt_h, tf_h, a1_h, c1_h, d1_h, et_h, te_h, out_h,
             A, B, C, D, Z, IT, FT, ACC, STG, sem0, sem1):
    c = lax.axis_index("c")
    s = lax.axis_index("s")
    zero = jnp.zeros((16,), jnp.float32)
    for r in range(48):
        ACC[pl.ds(r * 16, 16)] = zero

    def acc_add(r, v):
        ACC[pl.ds(r * 16, 16)] = ACC[pl.ds(r * 16, 16)] + v

    def colsum(base):
        v = zero
        for j in range(16):
            v = v + plsc.load_gather(
                ACC, [lax.iota(jnp.int32, 16) * 16 + (base * 16 + j)])
        return v

    @pl.when(c == 0)
    def _tet():
        pltpu.sync_copy(a0_h.at[s], A)
        pltpu.sync_copy(c0_h.at[s], C)

        def tt_sweep(gather_fn, store_fn):
            def chunk(cc, _):
                pltpu.sync_copy(tt_h.at[cc], IT.at[pl.ds(0, 21 * FCL)])

                def grp(gg, _):
                    accs = None
                    for k in range(21):
                        idx = IT[pl.ds(k * FCL + gg * 16, 16)]
                        vals = gather_fn(idx)
                        accs = vals if accs is None else [a + v for a, v in zip(accs, vals)]
                    store_fn(cc * FG + gg, accs)
                    return 0
                lax.fori_loop(0, FG, grp, 0)
                return 0
            lax.fori_loop(0, FCH, chunk, 0)

        def tf_sweep(nrows, upd_fn):
            def chunk(cq, _):
                pltpu.sync_copy(tf_h.at[cq], FT.at[pl.ds(0, 4 * QCL)])

                def grp(gg, _):
                    sl = [zero] * nrows
                    for f in range(4):
                        idx = FT[pl.ds(f * QCL + gg * 16, 16)]
                        for r in range(nrows):
                            sl[r] = sl[r] + plsc.load_gather(Z, [idx + (r * NTP)])
                    upd_fn(cq * QG + gg, sl)
                    return 0
                lax.fori_loop(0, QG, grp, 0)
                return 0
            lax.fori_loop(0, QCH, chunk, 0)

        def zstore0(fg, accs):
            Z[pl.ds(fg * 16, 16)] = accs[0]
        tt_sweep(lambda idx: [plsc.load_gather(C, [idx])], zstore0)

        def deg_upd(tq, sl):
            qvv = C[pl.ds(tq * 16, 16)]
            deg = qvv * (sl[0] - 4.0 * qvv)
            D[pl.ds(tq * 16, 16)] = jnp.where(deg > 0.0, 1.0 / deg, 0.0)
            acc_add(15, qvv)
            for d in range(DIM):
                acc_add(d, qvv * A[pl.ds(d * NQP + tq * 16, 16)])
        tf_sweep(1, deg_upd)

        for k in range(1, 9):
            def upd(tq, sl, k=k):
                qvv = C[pl.ds(tq * 16, 16)]
                w = D[pl.ds(tq * 16, 16)]
                for d in range(DIM):
                    xv = A[pl.ds(d * NQP + tq * 16, 16)]
                    nv = 0.5 * xv + 0.5 * qvv * (sl[d] - 4.0 * w * xv)
                    A[pl.ds(d * NQP + tq * 16, 16)] = nv
                    if k in POOL_DIFF:
                        dv = jnp.abs(B[pl.ds(d * NQP + tq * 16, 16)] - nv)
                        acc_add(POOL_DIFF[k] + d, qvv * dv)
                    if k == 8:
                        acc_add(12 + d, qvv * nv)
                    if k in SNAP:
                        B[pl.ds(d * NQP + tq * 16, 16)] = nv

            def zgather(idx):
                w = plsc.load_gather(D, [idx])
                return [w * plsc.load_gather(A, [idx + (d * NQP)])
                        for d in range(DIM)]

            def zstore(fg, accs):
                for d in range(DIM):
                    Z[pl.ds(d * NTP + fg * 16, 16)] = accs[d]
            tt_sweep(zgather, zstore)
            tf_sweep(DIM, upd)

        STG[pl.ds(48, 16)] = colsum(0)
        pltpu.sync_copy(STG.at[pl.ds(48, 16)], out_h.at[s, pl.ds(48, 16)])

    @pl.when(c == 1)
    def _tri():
        pltpu.sync_copy(a1_h.at[s], A)
        pltpu.sync_copy(c1_h.at[s], C)
        pltpu.sync_copy(d1_h.at[s], D)
        pltpu.sync_copy(et_h, IT.at[pl.ds(0, 22 * NEP)])
        pltpu.sync_copy(te_h, FT.at[pl.ds(0, 3 * NTP)])

        ACC[pl.ds(31 * 16, 16)] = jnp.full((16,), 1.5, jnp.float32)

        def egrp(gg, _):
            emv = C[pl.ds(EM_OFF + gg * 16, 16)]
            acc_add(47, emv)
            for d in range(DIM):
                xev = A[pl.ds(d * NQP + XE_OFF + gg * 16, 16)]
                axv = jnp.abs(xev)
                acc_add(32 + d, emv * xev)
                acc_add(35 + d, 0.25 * emv * axv)
                acc_add(38 + d, 0.1875 * emv * axv)
                acc_add(41 + d, 0.05859375 * emv * axv)
                acc_add(44 + d, (0.5 ** 8) * emv * xev)
            return 0
        lax.fori_loop(0, 36, egrp, 0)

        def et_sweep(gather_fn, store_fn):
            def grp(gg, _):
                accs = None
                for m in range(22):
                    idx = IT[pl.ds(m * NEP + gg * 16, 16)]
                    vals = gather_fn(idx)
                    accs = vals if accs is None else [a + v for a, v in zip(accs, vals)]
                store_fn(gg, accs)
                return 0
            lax.fori_loop(0, NEP // 16, grp, 0)

        def te_sweep(nrows, upd_fn):
            def grp(gg, _):
                sl = [zero] * nrows
                for f in range(3):
                    idx = FT[pl.ds(f * NTP + gg * 16, 16)]
                    for r in range(nrows):
                        sl[r] = sl[r] + plsc.load_gather(
                            B, [idx + (r * NQP + ZE_OFF)])
                upd_fn(gg, sl)
                return 0
            lax.fori_loop(0, NTP // 16, grp, 0)

        def zstore0(gg, accs):
            B[pl.ds(ZE_OFF + gg * 16, 16)] = accs[0]
        et_sweep(lambda idx: [plsc.load_gather(C, [idx])], zstore0)

        def tdeg_upd(tg, sl):
            tvv = C[pl.ds(tg * 16, 16)]
            deg = 2.0 * tvv * (sl[0] - 3.0 * tvv)
            D[pl.ds(tg * 16, 16)] = jnp.where(deg > 0.0, 1.0 / deg, 0.0)
            acc_add(15, tvv)
            for d in range(DIM):
                acc_add(d, tvv * A[pl.ds(d * NQP + tg * 16, 16)])
        te_sweep(1, tdeg_upd)

        for gg in range(2):
            for d in range(DIM):
                acc_add(16 + d, A[pl.ds(d * NQP + XN_OFF + gg * 16, 16)])

        for k in range(1, 9):
            def zgather(idx):
                w = plsc.load_gather(D, [idx])
                return [w * plsc.load_gather(A, [idx + (d * NQP)])
                        for d in range(DIM)]

            def zstore(gg, accs):
                for d in range(DIM):
                    B[pl.ds(d * NQP + ZE_OFF + gg * 16, 16)] = accs[d]
            et_sweep(zgather, zstore)

            def upd(tg, sl, k=k):
                tvv = C[pl.ds(tg * 16, 16)]
                w = D[pl.ds(tg * 16, 16)]
                for d in range(DIM):
                    xv = A[pl.ds(d * NQP + tg * 16, 16)]
                    nv = 0.5 * xv + tvv * (sl[d] - 3.0 * w * xv)
                    A[pl.ds(d * NQP + tg * 16, 16)] = nv
                    if k in POOL_DIFF:
                        dv = jnp.abs(B[pl.ds(d * NQP + tg * 16, 16)] - nv)
                        acc_add(POOL_DIFF[k] + d, tvv * dv)
                    if k == 8:
                        acc_add(12 + d, tvv * nv)
                    if k in SNAP:
                        B[pl.ds(d * NQP + tg * 16, 16)] = nv
            te_sweep(DIM, upd)

            for gg in range(2):
                dinv_ = C[pl.ds(DIN_OFF + gg * 16, 16)]
                for d in range(DIM):
                    B[pl.ds(d * NQP + YN_OFF + gg * 16, 16)] = (
                        dinv_ * A[pl.ds(d * NQP + XN_OFF + gg * 16, 16)])
            for d in range(DIM):
                def jloop(j, carry, d=d):
                    a0, a1 = carry
                    y = plsc.load_gather(
                        B, [jnp.full((16,), d * NQP + YN_OFF, jnp.int32) + j])
                    a0 = a0 + y * D[pl.ds(WEC_OFF + j * 32, 16)]
                    a1 = a1 + y * D[pl.ds(WEC_OFF + j * 32 + 16, 16)]
                    return (a0, a1)
                an0, an1 = lax.fori_loop(0, N, jloop, (zero, zero))
                for gg, an in ((0, an0), (1, an1)):
                    xv = A[pl.ds(d * NQP + XN_OFF + gg * 16, 16)]
                    nv = 0.5 * (xv + an)
                    A[pl.ds(d * NQP + XN_OFF + gg * 16, 16)] = nv
                    if k in POOL_DIFF:
                        dv = jnp.abs(
                            B[pl.ds(d * NQP + PRN_OFF + gg * 16, 16)] - nv)
                        acc_add(16 + POOL_DIFF[k] + d, dv)
                    if k == 8:
                        acc_add(16 + 12 + d, nv)
                    if k in SNAP:
                        B[pl.ds(d * NQP + PRN_OFF + gg * 16, 16)] = nv

        STG[pl.ds(0, 16)] = colsum(16)
        STG[pl.ds(16, 16)] = colsum(32)
        STG[pl.ds(32, 16)] = colsum(0)
        pltpu.sync_copy(STG.at[pl.ds(0, 48)], out_h.at[s, pl.ds(0, 48)])


_sc_call = functools.partial(
    pl.kernel,
    out_type=jax.ShapeDtypeStruct((16, 64), jnp.float32),
    mesh=plsc.VectorSubcoreMesh(core_axis_name="c", subcore_axis_name="s"),
    compiler_params=pltpu.CompilerParams(needs_layout_passes=False),
    scratch_types=[
        pltpu.VMEM((DIM * NQP,), jnp.float32),
        pltpu.VMEM((DIM * NQP,), jnp.float32),
        pltpu.VMEM((NQP,), jnp.float32),
        pltpu.VMEM((NQP,), jnp.float32),
        pltpu.VMEM((DIM * NTP,), jnp.float32),
        pltpu.VMEM((21 * FCL,), jnp.int32),
        pltpu.VMEM((4 * QCL,), jnp.int32),
        pltpu.VMEM((64 * 16,), jnp.float32),
        pltpu.VMEM((64,), jnp.float32),
        pltpu.SemaphoreType.DMA,
        pltpu.SemaphoreType.DMA,
    ],
)(_sc_body)


def kernel(point_clouds, alphas, sigma):
    pc = point_clouds.astype(jnp.float32)
    al = alphas.astype(jnp.float32)
    sig = jnp.asarray(sigma, dtype=jnp.float32)
    B_, Np, Dm = pc.shape
    G = B_ * al.shape[0]

    X = (pc[:, None, :, :] * al[None, :, None, :]).reshape(G, Np, Dm)
    Gm = jnp.einsum('gnd,gmd->gnm', X, X)
    dg = jnp.diagonal(Gm, axis1=1, axis2=2)
    Dmat = dg[:, None, :] + dg[:, :, None] - 2.0 * Gm
    Wm = jnp.exp(-Dmat / sig)
    adj = (Wm >= THRESH).astype(jnp.float32)
    We = Wm * adj
    em = adj.reshape(G, Np * Np)

    t0, t1, t2 = _TRI_NP[:, 0], _TRI_NP[:, 1], _TRI_NP[:, 2]
    af = em
    tvr = af[:, t0 * Np + t1] * af[:, t0 * Np + t2] * af[:, t1 * Np + t2]
    cnt = jnp.cumsum(tvr.astype(jnp.int32), axis=1)
    tv = tvr * (cnt <= MAXTRI).astype(jnp.float32)
    q0, q1, q2, q3 = (_TET_NP[:, 0], _TET_NP[:, 1], _TET_NP[:, 2], _TET_NP[:, 3])
    qv = (af[:, q0 * Np + q1] * af[:, q0 * Np + q2] * af[:, q0 * Np + q3]
          * af[:, q1 * Np + q2] * af[:, q1 * Np + q3] * af[:, q2 * Np + q3])

    Xe = 0.5 * (X[:, _IIX, :] + X[:, _JJX, :])
    Xt = (X[:, t0, :] + X[:, t1, :] + X[:, t2, :]) / 3.0
    Xq = (X[:, q0, :] + X[:, q1, :] + X[:, q2, :] + X[:, q3, :]) / 4.0

    deg_n = We.sum(axis=1)
    deg_n = deg_n.at[0].add(We.sum(axis=(0, 1)))
    di_n = jnp.where(deg_n > 0, 1.0 / deg_n, 0.0)
    wec = jnp.transpose(We, (0, 2, 1))
    wec = wec.at[0].add(We.sum(axis=0).T)
    wec = jnp.pad(wec, ((0, 0), (0, 0), (0, 32 - Np))).reshape(G, N * 32)

    def pad_to(x, n):
        return jnp.pad(x, ((0, 0), (0, 0), (0, n - x.shape[-1])))

    a0 = pad_to(jnp.transpose(Xq, (0, 2, 1)), NQP).reshape(G, DIM * NQP)
    c0 = jnp.pad(qv, ((0, 0), (0, NQP - NQ)))
    a1 = jnp.concatenate([
        pad_to(jnp.transpose(Xt, (0, 2, 1)), NTP),
        jnp.transpose(Xe, (0, 2, 1)),
        pad_to(jnp.transpose(X, (0, 2, 1)), 32),
        jnp.zeros((G, DIM, NQP - XN_OFF - 32), jnp.float32),
    ], axis=-1).reshape(G, DIM * NQP)
    c1 = jnp.concatenate([
        jnp.pad(tv, ((0, 0), (0, NTP - NT))), em,
        jnp.pad(di_n, ((0, 0), (0, 32 - Np))),
        jnp.zeros((G, NQP - DIN_OFF - 32), jnp.float32),
    ], axis=-1)
    d1 = jnp.concatenate([
        jnp.zeros((G, WEC_OFF), jnp.float32), wec,
        jnp.zeros((G, NQP - WEC_OFF - N * 32), jnp.float32),
    ], axis=-1)

    out = _sc_call(a0, c0, jnp.asarray(_TTC), jnp.asarray(_TFC),
                   a1, c1, d1, jnp.asarray(_ETF), jnp.asarray(_TEF))

    out = out.reshape(G, 4, 16)
    sums = out[:, :, :15].sum(axis=1)
    cnts = out[:, :, 15].sum(axis=1)
    pooled = sums / jnp.maximum(cnts, 1.0)[:, None]
    return pooled.reshape(B_, -1)
